# Initial kernel scaffold; baseline (speedup 1.0000x reference)
import jax
import jax.numpy as jnp
from jax.experimental import pallas as pl


def kernel(flow0, flow1, flow2, flow3, events, frame, frame_, num_events):
    raise NotImplementedError("write your pallas kernel here")


# speedup vs baseline: 7.7010x; 7.7010x over previous
import jax
import jax.numpy as jnp
from jax import lax
from jax.experimental import pallas as pl
from jax.experimental.pallas import tpu as pltpu
from jax.experimental.pallas import tpu_sc as plsc

_B = 2
_N = 100000
_H = 512
_W = 512
_EPS = float(jnp.finfo(jnp.float32).eps)

_NT = 16
_NCH = 49
_CHUNK = _NCH * 128
_NV = _CHUNK // 16
_NPAD = _NT * _CHUNK
_GMAX = 65536
_RCH = 1024


def _f16(x, dtype=jnp.float32):
    return jnp.full((16,), x, dtype)


def _make_scale_body(scale):
    w = _W >> (3 - scale)
    h = _H >> (3 - scale)
    p_grid = w * h
    sh = 3 - scale
    halves = max(1, p_grid // _GMAX)
    hp = p_grid // halves

    def body(fx_tab, fy_tab, ch, th, out,
             ev_c, ev_t, lin, fxb, fyb,
             idxa, idxb, idxc, idxd, numa, numb, numc, numd,
             mfa, mfb2, mfc, mfd,
             gbuf_d, gbuf_n, zbuf, tbufa, tbufb,
             stat_stage, stat_all, acc_buf, part_all,
             den_sh, num_sh, stat_sh, part_sh):
        cid = lax.axis_index("c")
        sid = lax.axis_index("s")
        base = sid * _CHUNK

        pltpu.sync_copy(ch.at[cid, pl.ds(base, _CHUNK)], ev_c)
        pltpu.sync_copy(th.at[cid, pl.ds(base, _CHUNK)], ev_t)
        pltpu.sync_copy(th.at[cid, pl.ds(0, 16)], tbufa)
        pltpu.sync_copy(th.at[cid, pl.ds(_NPAD - 16, 16)], tbufb)
        t_first = tbufa[...][0]
        t_last = tbufb[...][15]

        def _fill_zero(i, _):
            zbuf[pl.ds(i * 16, 16)] = jnp.zeros((16,), jnp.float32)
            return 0
        lax.fori_loop(0, _RCH // 16, _fill_zero, 0)
        zper = hp // _NT
        zcsz = min(zper, _RCH)
        for c in range(zper // zcsz):
            off = sid * zper + c * zcsz
            pltpu.sync_copy(zbuf.at[pl.ds(0, zcsz)],
                            den_sh.at[pl.ds(off, zcsz)])
            pltpu.sync_copy(zbuf.at[pl.ds(0, zcsz)],
                            num_sh.at[pl.ds(off, zcsz)])

        ninf = _f16(-jnp.inf)
        pinf = _f16(jnp.inf)

        def _stat(i, carry):
            mxp, mnp, mxn, mnn = carry
            sl = pl.ds(i * 16, 16)
            tv = ev_t[sl]
            pc = lax.shift_right_logical(ev_c[sl], 20)
            mp = pc == 1
            mn = pc == 2
            mxp = jnp.maximum(mxp, jnp.where(mp, tv, ninf))
            mnp = jnp.minimum(mnp, jnp.where(mp, tv, pinf))
            mxn = jnp.maximum(mxn, jnp.where(mn, tv, ninf))
            mnn = jnp.minimum(mnn, jnp.where(mn, tv, pinf))
            return (mxp, mnp, mxn, mnn)

        mxp, mnp, mxn, mnn = lax.fori_loop(0, _NV, _stat,
                                           (ninf, pinf, ninf, pinf))
        stat_stage[0, :] = mxp
        stat_stage[1, :] = mnp
        stat_stage[2, :] = mxn
        stat_stage[3, :] = mnn
        pltpu.sync_copy(stat_stage, stat_sh.at[sid])
        plsc.subcore_barrier()
        pltpu.sync_copy(stat_sh, stat_all)
        vmxp, vmnp, vmxn, vmnn = (stat_all[0, 0, :], stat_all[0, 1, :],
                                  stat_all[0, 2, :], stat_all[0, 3, :])
        for i in range(1, _NT):
            vmxp = jnp.maximum(vmxp, stat_all[i, 0, :])
            vmnp = jnp.minimum(vmnp, stat_all[i, 1, :])
            vmxn = jnp.maximum(vmxn, stat_all[i, 2, :])
            vmnn = jnp.minimum(vmnn, stat_all[i, 3, :])

        tmaxp = _f16(jnp.max(vmxp))
        tminp = _f16(jnp.min(vmnp))
        tmaxn = _f16(jnp.max(vmxn))
        tminn = _f16(jnp.min(vmnn))
        tfv = _f16(t_first)
        tlv = _f16(t_last)

        u = tlv
        for step in range(scale + 1):
            if step == 0:
                sv = tfv
                dv = (tlv - tfv) + _EPS
            else:
                sv = jnp.zeros((16,), jnp.float32)
                dv = u + _EPS

            def _tnorm(k, _, sv=sv, dv=dv):
                sl = pl.ds(k * 16, 16)
                ev_t[sl] = (ev_t[sl] - sv) / dv
                return 0
            lax.fori_loop(0, _NV, _tnorm, 0)
            tmaxp = (tmaxp - sv) / dv
            tminp = (tminp - sv) / dv
            tmaxn = (tmaxn - sv) / dv
            tminn = (tminn - sv) / dv
            u = (u - sv) / dv

        boff = cid * p_grid
        m511 = jnp.full((16,), 511, jnp.int32)

        def _lin(j, _):
            for l in range(8):
                sl = pl.ds(j * 128 + l * 16, 16)
                cv = ev_c[sl]
                xs = lax.shift_right_logical(cv & m511, sh)
                ys = lax.shift_right_logical(
                    lax.shift_right_logical(cv, 9) & m511, sh)
                lin[pl.ds(j * 128 + l * 16, 16)] = ys * w + xs + boff
            return 0
        lax.fori_loop(0, _NCH, _lin, 0)

        def _gath(j, _):
            sl = pl.ds(j * 128, 128)
            pltpu.sync_copy(fx_tab.at[lin.at[sl]], fxb.at[sl])
            pltpu.sync_copy(fy_tab.at[lin.at[sl]], fyb.at[sl])
            return 0
        lax.fori_loop(0, _NCH, _gath, 0)

        acc = jnp.zeros((16,), jnp.float32)

        def _pass(pi, acc):
            pos_v = jnp.full((16,), pi < 2)
            fwd_v = jnp.full((16,), (pi == 0) | (pi == 2))
            one = jnp.full((16,), 1.0, jnp.float32)
            zero = jnp.zeros((16,), jnp.float32)
            polsel = jnp.where(pos_v, jnp.full((16,), 1, jnp.int32),
                               jnp.full((16,), 2, jnp.int32))
            tev = jnp.where(fwd_v,
                            jnp.where(pos_v, tmaxp, tmaxn),
                            jnp.where(pos_v, tminp, tminn))
            sev = jnp.where(fwd_v, _f16(_EPS), _f16(-_EPS))

            for hh in range(halves):
                lo = hh * hp

                def _stage(j, _, lo=lo):
                    for l in range(8):
                        sl = pl.ds(j * 128 + l * 16, 16)
                        sl2 = pl.ds(l * 16, 16)
                        tv = ev_t[sl]
                        cv = ev_c[sl]
                        pc = lax.shift_right_logical(cv, 20)
                        m = pc == polsel
                        t_ = (tev - tv) + sev
                        xs = lax.shift_right_logical(cv & m511, sh)
                        ys = lax.shift_right_logical(
                            lax.shift_right_logical(cv, 9) & m511, sh)
                        xsf = xs.astype(jnp.float32)
                        ysf = ys.astype(jnp.float32)
                        fxv = fxb[sl]
                        fyv = fyb[sl]
                        x_ = jnp.minimum(jnp.maximum(xsf + t_ * fxv, 0.0),
                                         float(w - 1))
                        y_ = jnp.minimum(jnp.maximum(ysf + t_ * fyv, 0.0),
                                         float(h - 1))
                        x0i = x_.astype(jnp.int32)
                        y0i = y_.astype(jnp.int32)
                        x0f = x0i.astype(jnp.float32)
                        y0f = y0i.astype(jnp.float32)
                        cx = x_ > x0f
                        cy = y_ > y0f
                        x1f = jnp.where(cx, x0f + 1.0, x0f)
                        y1f = jnp.where(cy, y0f + 1.0, y0f)
                        x1i = x0i + cx.astype(jnp.int32)
                        y1i = y0i + cy.astype(jnp.int32)
                        x0r = 1.0 - (x_ - x0f)
                        x1r = 1.0 - (x1f - x_)
                        y0r = 1.0 - (y_ - y0f)
                        y1r = 1.0 - (y1f - y_)
                        ra = x0r * y0r + _EPS
                        rb = x1r * y0r + _EPS
                        rc = x0r * y1r + _EPS
                        rd = x1r * y1r + _EPS
                        y0w = y0i * w
                        y1w = y1i * w
                        hm1 = hp - 1
                        ia = x0i + y0w - lo
                        ib = x1i + y0w - lo
                        ic = x0i + y1w - lo
                        idd = x1i + y1w - lo
                        for ibx, rx, idst, vdst, mdst in (
                                (ia, ra, idxa, numa, mfa),
                                (ib, rb, idxb, numb, mfb2),
                                (ic, rc, idxc, numc, mfc),
                                (idd, rd, idxd, numd, mfd)):
                            inh = m & (ibx >= 0) & (ibx < hp)
                            idst[sl2] = jnp.minimum(
                                jnp.maximum(ibx, 0), hm1)
                            vdst[sl2] = jnp.where(inh, rx * t_, zero)
                            mdst[sl2] = jnp.where(inh, one, zero)
                    for idst, vdst, mdst in ((idxa, numa, mfa),
                                             (idxb, numb, mfb2),
                                             (idxc, numc, mfc),
                                             (idxd, numd, mfd)):
                        pltpu.sync_copy(mdst, den_sh.at[idst], add=True)
                        pltpu.sync_copy(vdst, num_sh.at[idst], add=True)
                    return 0

                lax.fori_loop(0, _NCH, _stage, 0)
                plsc.subcore_barrier()

                slice_p = hp // _NT
                csz = min(slice_p, _RCH)
                for c in range(slice_p // csz):
                    off = sid * slice_p + c * csz
                    pltpu.sync_copy(den_sh.at[pl.ds(off, csz)],
                                    gbuf_d.at[pl.ds(0, csz)])
                    pltpu.sync_copy(num_sh.at[pl.ds(off, csz)],
                                    gbuf_n.at[pl.ds(0, csz)])

                    def _red(k, a):
                        sl = pl.ds(k * 16, 16)
                        r = gbuf_n[sl] / (gbuf_d[sl] + _EPS)
                        return a + r * r
                    acc = lax.fori_loop(0, csz // 16, _red, acc)
                    pltpu.sync_copy(zbuf.at[pl.ds(0, csz)],
                                    den_sh.at[pl.ds(off, csz)])
                    pltpu.sync_copy(zbuf.at[pl.ds(0, csz)],
                                    num_sh.at[pl.ds(off, csz)])
                plsc.subcore_barrier()
            return acc

        acc = lax.fori_loop(0, 4, _pass, acc)

        acc_buf[...] = acc
        pltpu.sync_copy(acc_buf, part_sh.at[sid])
        plsc.subcore_barrier()

        @pl.when(sid == 0)
        def _():
            pltpu.sync_copy(part_sh, part_all)
            tot = part_all[0, :]
            for i in range(1, _NT):
                tot = tot + part_all[i, :]
            acc_buf[...] = tot
            pltpu.sync_copy(acc_buf, out.at[cid])

    return body, hp


def _event_loss_sc(flows, events):
    x = events[:, 0, :]
    y = events[:, 1, :]
    t = events[:, 2, :]
    p = events[:, 3, :]
    xi = x.astype(jnp.int32)
    yi = y.astype(jnp.int32)
    pc = jnp.where(p == 1.0, 1, 0) + jnp.where(p == -1.0, 2, 0)
    code = xi | (yi << 9) | (pc << 20)
    pad = _NPAD - _N
    ch = jnp.pad(code, ((0, 0), (0, pad)))
    th = jnp.pad(t, ((0, 0), (0, pad)), mode="edge")

    mesh = plsc.VectorSubcoreMesh(core_axis_name="c", subcore_axis_name="s")
    ev = jnp.float32(0.0)
    for i, f in enumerate(flows):
        bb, _, hh, ww = f.shape
        fx_tab = f[:, 0].reshape(bb * hh * ww)
        fy_tab = f[:, 1].reshape(bb * hh * ww)
        body, gsz = _make_scale_body(i)
        run = pl.kernel(
            body,
            out_type=jax.ShapeDtypeStruct((_B, 16), jnp.float32),
            mesh=mesh,
            compiler_params=pltpu.CompilerParams(needs_layout_passes=False),
            scratch_types=[
                pltpu.VMEM((_CHUNK,), jnp.int32),
                pltpu.VMEM((_CHUNK,), jnp.float32),
                pltpu.VMEM((_CHUNK,), jnp.int32),
                pltpu.VMEM((_CHUNK,), jnp.float32),
                pltpu.VMEM((_CHUNK,), jnp.float32),
                pltpu.VMEM((128,), jnp.int32),
                pltpu.VMEM((128,), jnp.int32),
                pltpu.VMEM((128,), jnp.int32),
                pltpu.VMEM((128,), jnp.int32),
                pltpu.VMEM((128,), jnp.float32),
                pltpu.VMEM((128,), jnp.float32),
                pltpu.VMEM((128,), jnp.float32),
                pltpu.VMEM((128,), jnp.float32),
                pltpu.VMEM((128,), jnp.float32),
                pltpu.VMEM((128,), jnp.float32),
                pltpu.VMEM((128,), jnp.float32),
                pltpu.VMEM((128,), jnp.float32),
                pltpu.VMEM((_RCH,), jnp.float32),
                pltpu.VMEM((_RCH,), jnp.float32),
                pltpu.VMEM((_RCH,), jnp.float32),
                pltpu.VMEM((16,), jnp.float32),
                pltpu.VMEM((16,), jnp.float32),
                pltpu.VMEM((4, 16), jnp.float32),
                pltpu.VMEM((_NT, 4, 16), jnp.float32),
                pltpu.VMEM((16,), jnp.float32),
                pltpu.VMEM((_NT, 16), jnp.float32),
                pltpu.VMEM_SHARED((gsz,), jnp.float32),
                pltpu.VMEM_SHARED((gsz,), jnp.float32),
                pltpu.VMEM_SHARED((_NT, 4, 16), jnp.float32),
                pltpu.VMEM_SHARED((_NT, 16), jnp.float32),
            ],
        )
        parts = run(fx_tab, fy_tab, ch, th)
        ev = ev + jnp.sum(parts)
    return ev


def _smooth_body(f0, f1, f2, f3, out):
    tot = jnp.float32(0.0)
    for ref in (f0, f1, f2, f3):
        f = ref[...]
        d1 = f[:, :, 1:, :] - f[:, :, :-1, :]
        d2 = f[:, :, :, 1:] - f[:, :, :, :-1]
        d3 = f[:, :, 1:, 1:] - f[:, :, :-1, :-1]
        d4 = f[:, :, :-1, 1:] - f[:, :, 1:, :-1]
        for d in (d1, d2, d3, d4):
            sq = d * d + jnp.float32(1e-6)
            tot = tot + jnp.mean(jnp.exp(jnp.float32(0.45) * jnp.log(sq)))
    out[...] = jnp.full((8, 128), tot * jnp.float32(0.5), jnp.float32)


def _smoothness_tc(flows):
    return pl.pallas_call(
        _smooth_body,
        out_shape=jax.ShapeDtypeStruct((8, 128), jnp.float32),
    )(*flows)[0, 0]


def kernel(flow0, flow1, flow2, flow3, events, frame, frame_, num_events):
    flows = (flow0, flow1, flow2, flow3)
    ev = _event_loss_sc(flows, events)
    smooth = _smoothness_tc(flows)
    loss = ev + smooth
    return (loss, ev, smooth)


# speedup vs baseline: 8.0147x; 1.0407x over previous
import jax
import jax.numpy as jnp
from jax import lax
from jax.experimental import pallas as pl
from jax.experimental.pallas import tpu as pltpu
from jax.experimental.pallas import tpu_sc as plsc

_B = 2
_N = 100000
_H = 512
_W = 512
_EPS = float(jnp.finfo(jnp.float32).eps)

_NT = 16
_NCH = 49
_CHUNK = _NCH * 128
_NV = _CHUNK // 16
_NPAD = _NT * _CHUNK
_GMAX = 65536
_RCH = 1024


def _f16(x, dtype=jnp.float32):
    return jnp.full((16,), x, dtype)


def _make_scale_body(scale):
    w = _W >> (3 - scale)
    h = _H >> (3 - scale)
    p_grid = w * h
    sh = 3 - scale
    halves = max(1, p_grid // _GMAX)
    hp = p_grid // halves

    def body(fx_tab, fy_tab, ch, th, out,
             ev_c, ev_t, lin, fxb, fyb,
             idxa, idxb, idxc, idxd, numa, numb, numc, numd,
             mfa, mfb2, mfc, mfd,
             gbuf_d, gbuf_n, zbuf, dummy, tbufa, tbufb,
             stat_stage, stat_all, acc_buf, part_all,
             den_sh, num_sh, stat_sh, part_sh, sem_g, sem_s):
        cid = lax.axis_index("c")
        sid = lax.axis_index("s")
        base = sid * _CHUNK

        pltpu.sync_copy(ch.at[cid, pl.ds(base, _CHUNK)], ev_c)
        pltpu.sync_copy(th.at[cid, pl.ds(base, _CHUNK)], ev_t)
        pltpu.sync_copy(th.at[cid, pl.ds(0, 16)], tbufa)
        pltpu.sync_copy(th.at[cid, pl.ds(_NPAD - 16, 16)], tbufb)
        t_first = tbufa[...][0]
        t_last = tbufb[...][15]

        def _fill_zero(i, _):
            zbuf[pl.ds(i * 16, 16)] = jnp.zeros((16,), jnp.float32)
            return 0
        lax.fori_loop(0, _RCH // 16, _fill_zero, 0)
        zper = hp // _NT
        zcsz = min(zper, _RCH)
        for c in range(zper // zcsz):
            off = sid * zper + c * zcsz
            pltpu.sync_copy(zbuf.at[pl.ds(0, zcsz)],
                            den_sh.at[pl.ds(off, zcsz)])
            pltpu.sync_copy(zbuf.at[pl.ds(0, zcsz)],
                            num_sh.at[pl.ds(off, zcsz)])

        ninf = _f16(-jnp.inf)
        pinf = _f16(jnp.inf)

        def _stat(i, carry):
            mxp, mnp, mxn, mnn = carry
            sl = pl.ds(i * 16, 16)
            tv = ev_t[sl]
            pc = lax.shift_right_logical(ev_c[sl], 20)
            mp = pc == 1
            mn = pc == 2
            mxp = jnp.maximum(mxp, jnp.where(mp, tv, ninf))
            mnp = jnp.minimum(mnp, jnp.where(mp, tv, pinf))
            mxn = jnp.maximum(mxn, jnp.where(mn, tv, ninf))
            mnn = jnp.minimum(mnn, jnp.where(mn, tv, pinf))
            return (mxp, mnp, mxn, mnn)

        mxp, mnp, mxn, mnn = lax.fori_loop(0, _NV, _stat,
                                           (ninf, pinf, ninf, pinf))
        stat_stage[0, :] = mxp
        stat_stage[1, :] = mnp
        stat_stage[2, :] = mxn
        stat_stage[3, :] = mnn
        pltpu.sync_copy(stat_stage, stat_sh.at[sid])
        plsc.subcore_barrier()
        pltpu.sync_copy(stat_sh, stat_all)
        vmxp, vmnp, vmxn, vmnn = (stat_all[0, 0, :], stat_all[0, 1, :],
                                  stat_all[0, 2, :], stat_all[0, 3, :])
        for i in range(1, _NT):
            vmxp = jnp.maximum(vmxp, stat_all[i, 0, :])
            vmnp = jnp.minimum(vmnp, stat_all[i, 1, :])
            vmxn = jnp.maximum(vmxn, stat_all[i, 2, :])
            vmnn = jnp.minimum(vmnn, stat_all[i, 3, :])

        tmaxp = _f16(jnp.max(vmxp))
        tminp = _f16(jnp.min(vmnp))
        tmaxn = _f16(jnp.max(vmxn))
        tminn = _f16(jnp.min(vmnn))
        tfv = _f16(t_first)
        tlv = _f16(t_last)

        u = tlv
        for step in range(scale + 1):
            if step == 0:
                sv = tfv
                dv = (tlv - tfv) + _EPS
            else:
                sv = jnp.zeros((16,), jnp.float32)
                dv = u + _EPS

            def _tnorm(k, _, sv=sv, dv=dv):
                sl = pl.ds(k * 16, 16)
                ev_t[sl] = (ev_t[sl] - sv) / dv
                return 0
            lax.fori_loop(0, _NV, _tnorm, 0)
            tmaxp = (tmaxp - sv) / dv
            tminp = (tminp - sv) / dv
            tmaxn = (tmaxn - sv) / dv
            tminn = (tminn - sv) / dv
            u = (u - sv) / dv

        boff = cid * p_grid
        m511 = jnp.full((16,), 511, jnp.int32)

        def _lin(j, _):
            for l in range(8):
                sl = pl.ds(j * 128 + l * 16, 16)
                cv = ev_c[sl]
                xs = lax.shift_right_logical(cv & m511, sh)
                ys = lax.shift_right_logical(
                    lax.shift_right_logical(cv, 9) & m511, sh)
                lin[pl.ds(j * 128 + l * 16, 16)] = ys * w + xs + boff
            return 0
        lax.fori_loop(0, _NCH, _lin, 0)

        def _gath(j, _):
            sl = pl.ds(j * 128, 128)
            pltpu.async_copy(fx_tab.at[lin.at[sl]], fxb.at[sl], sem_g)
            pltpu.async_copy(fy_tab.at[lin.at[sl]], fyb.at[sl], sem_g)
            for _i in range(2):
                pltpu.make_async_copy(
                    th.at[0, pl.ds(0, 128)], dummy, sem_g).wait()
            return 0
        lax.fori_loop(0, _NCH, _gath, 0)

        acc = jnp.zeros((16,), jnp.float32)

        def _pass(pi, acc):
            pos_v = jnp.full((16,), pi < 2)
            fwd_v = jnp.full((16,), (pi == 0) | (pi == 2))
            one = jnp.full((16,), 1.0, jnp.float32)
            zero = jnp.zeros((16,), jnp.float32)
            polsel = jnp.where(pos_v, jnp.full((16,), 1, jnp.int32),
                               jnp.full((16,), 2, jnp.int32))
            tev = jnp.where(fwd_v,
                            jnp.where(pos_v, tmaxp, tmaxn),
                            jnp.where(pos_v, tminp, tminn))
            sev = jnp.where(fwd_v, _f16(_EPS), _f16(-_EPS))

            for hh in range(halves):
                lo = hh * hp

                def _stage(j, _, lo=lo):
                    for l in range(8):
                        sl = pl.ds(j * 128 + l * 16, 16)
                        sl2 = pl.ds(l * 16, 16)
                        tv = ev_t[sl]
                        cv = ev_c[sl]
                        pc = lax.shift_right_logical(cv, 20)
                        m = pc == polsel
                        t_ = (tev - tv) + sev
                        xs = lax.shift_right_logical(cv & m511, sh)
                        ys = lax.shift_right_logical(
                            lax.shift_right_logical(cv, 9) & m511, sh)
                        xsf = xs.astype(jnp.float32)
                        ysf = ys.astype(jnp.float32)
                        fxv = fxb[sl]
                        fyv = fyb[sl]
                        x_ = jnp.minimum(jnp.maximum(xsf + t_ * fxv, 0.0),
                                         float(w - 1))
                        y_ = jnp.minimum(jnp.maximum(ysf + t_ * fyv, 0.0),
                                         float(h - 1))
                        x0i = x_.astype(jnp.int32)
                        y0i = y_.astype(jnp.int32)
                        x0f = x0i.astype(jnp.float32)
                        y0f = y0i.astype(jnp.float32)
                        cx = x_ > x0f
                        cy = y_ > y0f
                        x1f = jnp.where(cx, x0f + 1.0, x0f)
                        y1f = jnp.where(cy, y0f + 1.0, y0f)
                        x1i = x0i + cx.astype(jnp.int32)
                        y1i = y0i + cy.astype(jnp.int32)
                        x0r = 1.0 - (x_ - x0f)
                        x1r = 1.0 - (x1f - x_)
                        y0r = 1.0 - (y_ - y0f)
                        y1r = 1.0 - (y1f - y_)
                        ra = x0r * y0r + _EPS
                        rb = x1r * y0r + _EPS
                        rc = x0r * y1r + _EPS
                        rd = x1r * y1r + _EPS
                        y0w = y0i * w
                        y1w = y1i * w
                        hm1 = hp - 1
                        ia = x0i + y0w - lo
                        ib = x1i + y0w - lo
                        ic = x0i + y1w - lo
                        idd = x1i + y1w - lo
                        for ibx, rx, idst, vdst, mdst in (
                                (ia, ra, idxa, numa, mfa),
                                (ib, rb, idxb, numb, mfb2),
                                (ic, rc, idxc, numc, mfc),
                                (idd, rd, idxd, numd, mfd)):
                            inh = m & (ibx >= 0) & (ibx < hp)
                            idst[sl2] = jnp.minimum(
                                jnp.maximum(ibx, 0), hm1)
                            vdst[sl2] = jnp.where(inh, rx * t_, zero)
                            mdst[sl2] = jnp.where(inh, one, zero)
                    for idst, vdst, mdst in ((idxa, numa, mfa),
                                             (idxb, numb, mfb2),
                                             (idxc, numc, mfc),
                                             (idxd, numd, mfd)):
                        pltpu.async_copy(mdst, den_sh.at[idst], sem_s,
                                         add=True)
                        pltpu.async_copy(vdst, num_sh.at[idst], sem_s,
                                         add=True)
                    for _ in range(8):
                        pltpu.make_async_copy(
                            th.at[0, pl.ds(0, 128)], dummy, sem_s).wait()
                    return 0

                lax.fori_loop(0, _NCH, _stage, 0)
                plsc.subcore_barrier()

                slice_p = hp // _NT
                csz = min(slice_p, _RCH)
                for c in range(slice_p // csz):
                    off = sid * slice_p + c * csz
                    pltpu.sync_copy(den_sh.at[pl.ds(off, csz)],
                                    gbuf_d.at[pl.ds(0, csz)])
                    pltpu.sync_copy(num_sh.at[pl.ds(off, csz)],
                                    gbuf_n.at[pl.ds(0, csz)])

                    def _red(k, a):
                        sl = pl.ds(k * 16, 16)
                        r = gbuf_n[sl] / (gbuf_d[sl] + _EPS)
                        return a + r * r
                    acc = lax.fori_loop(0, csz // 16, _red, acc)
                    pltpu.sync_copy(zbuf.at[pl.ds(0, csz)],
                                    den_sh.at[pl.ds(off, csz)])
                    pltpu.sync_copy(zbuf.at[pl.ds(0, csz)],
                                    num_sh.at[pl.ds(off, csz)])
                plsc.subcore_barrier()
            return acc

        acc = lax.fori_loop(0, 4, _pass, acc)

        acc_buf[...] = acc
        pltpu.sync_copy(acc_buf, part_sh.at[sid])
        plsc.subcore_barrier()

        @pl.when(sid == 0)
        def _():
            pltpu.sync_copy(part_sh, part_all)
            tot = part_all[0, :]
            for i in range(1, _NT):
                tot = tot + part_all[i, :]
            acc_buf[...] = tot
            pltpu.sync_copy(acc_buf, out.at[cid])

    return body, hp


def _event_loss_sc(flows, events):
    x = events[:, 0, :]
    y = events[:, 1, :]
    t = events[:, 2, :]
    p = events[:, 3, :]
    xi = x.astype(jnp.int32)
    yi = y.astype(jnp.int32)
    pc = jnp.where(p == 1.0, 1, 0) + jnp.where(p == -1.0, 2, 0)
    code = xi | (yi << 9) | (pc << 20)
    pad = _NPAD - _N
    ch = jnp.pad(code, ((0, 0), (0, pad)))
    th = jnp.pad(t, ((0, 0), (0, pad)), mode="edge")

    mesh = plsc.VectorSubcoreMesh(core_axis_name="c", subcore_axis_name="s")
    ev = jnp.float32(0.0)
    for i, f in enumerate(flows):
        bb, _, hh, ww = f.shape
        fx_tab = f[:, 0].reshape(bb * hh * ww)
        fy_tab = f[:, 1].reshape(bb * hh * ww)
        body, gsz = _make_scale_body(i)
        run = pl.kernel(
            body,
            out_type=jax.ShapeDtypeStruct((_B, 16), jnp.float32),
            mesh=mesh,
            compiler_params=pltpu.CompilerParams(needs_layout_passes=False),
            scratch_types=[
                pltpu.VMEM((_CHUNK,), jnp.int32),
                pltpu.VMEM((_CHUNK,), jnp.float32),
                pltpu.VMEM((_CHUNK,), jnp.int32),
                pltpu.VMEM((_CHUNK,), jnp.float32),
                pltpu.VMEM((_CHUNK,), jnp.float32),
                pltpu.VMEM((128,), jnp.int32),
                pltpu.VMEM((128,), jnp.int32),
                pltpu.VMEM((128,), jnp.int32),
                pltpu.VMEM((128,), jnp.int32),
                pltpu.VMEM((128,), jnp.float32),
                pltpu.VMEM((128,), jnp.float32),
                pltpu.VMEM((128,), jnp.float32),
                pltpu.VMEM((128,), jnp.float32),
                pltpu.VMEM((128,), jnp.float32),
                pltpu.VMEM((128,), jnp.float32),
                pltpu.VMEM((128,), jnp.float32),
                pltpu.VMEM((128,), jnp.float32),
                pltpu.VMEM((_RCH,), jnp.float32),
                pltpu.VMEM((_RCH,), jnp.float32),
                pltpu.VMEM((_RCH,), jnp.float32),
                pltpu.VMEM((128,), jnp.float32),
                pltpu.VMEM((16,), jnp.float32),
                pltpu.VMEM((16,), jnp.float32),
                pltpu.VMEM((4, 16), jnp.float32),
                pltpu.VMEM((_NT, 4, 16), jnp.float32),
                pltpu.VMEM((16,), jnp.float32),
                pltpu.VMEM((_NT, 16), jnp.float32),
                pltpu.VMEM_SHARED((gsz,), jnp.float32),
                pltpu.VMEM_SHARED((gsz,), jnp.float32),
                pltpu.VMEM_SHARED((_NT, 4, 16), jnp.float32),
                pltpu.VMEM_SHARED((_NT, 16), jnp.float32),
                pltpu.SemaphoreType.DMA,
                pltpu.SemaphoreType.DMA,
            ],
        )
        parts = run(fx_tab, fy_tab, ch, th)
        ev = ev + jnp.sum(parts)
    return ev


def _smooth_body(f0, f1, f2, f3, out):
    tot = jnp.float32(0.0)
    for ref in (f0, f1, f2, f3):
        f = ref[...]
        d1 = f[:, :, 1:, :] - f[:, :, :-1, :]
        d2 = f[:, :, :, 1:] - f[:, :, :, :-1]
        d3 = f[:, :, 1:, 1:] - f[:, :, :-1, :-1]
        d4 = f[:, :, :-1, 1:] - f[:, :, 1:, :-1]
        for d in (d1, d2, d3, d4):
            sq = d * d + jnp.float32(1e-6)
            tot = tot + jnp.mean(jnp.exp(jnp.float32(0.45) * jnp.log(sq)))
    out[...] = jnp.full((8, 128), tot * jnp.float32(0.5), jnp.float32)


def _smoothness_tc(flows):
    return pl.pallas_call(
        _smooth_body,
        out_shape=jax.ShapeDtypeStruct((8, 128), jnp.float32),
    )(*flows)[0, 0]


def kernel(flow0, flow1, flow2, flow3, events, frame, frame_, num_events):
    flows = (flow0, flow1, flow2, flow3)
    ev = _event_loss_sc(flows, events)
    smooth = _smoothness_tc(flows)
    loss = ev + smooth
    return (loss, ev, smooth)
